# baseline (speedup 1.0000x reference)
import jax
import jax.numpy as jnp
from jax.experimental import pallas as pl
from jax.experimental.pallas import tpu as pltpu

M_BLK = 512
H = M_BLK // 2
N = 64
HID = 32
NL = N + 3


def _ncm_block_kernel(atext_ref, order_ref, do_ref, u_ref,
                      w1t2_ref, wabs_ref, b2_ref,
                      out_ref, gx_ref):
    m_blk = atext_ref.shape[0]
    n = N
    order = order_ref[...]
    do_b = do_ref[...]
    u_b = u_ref[...]

    iota_k = jax.lax.broadcasted_iota(jnp.int32, (m_blk, n, n), 2)
    onehot = (order[:, :, None] == iota_k).astype(jnp.bfloat16)
    gx = jax.lax.dot_general(
        onehot, atext_ref[...],
        dimension_numbers=(((2,), (1,)), ((0,), (0,))),
        preferred_element_type=jnp.float32)
    gx_ref[...] = jax.lax.transpose(gx.astype(jnp.bfloat16), (1, 0, 2))

    lane = jax.lax.broadcasted_iota(jnp.int32, (m_blk, NL), 1)
    wabs = wabs_ref[...].astype(jnp.bfloat16)
    lanef1 = (jax.lax.broadcasted_iota(jnp.int32, (1, NL), 1)
              + 1).astype(jnp.bfloat16)

    outputs0 = jnp.where(lane == do_b, u_b,
                         jnp.where(lane >= n, 1.0, 0.0))
    not_do = lane != do_b

    ndA, ndB = not_do[:H], not_do[H:]

    def mm(ins):
        return jax.lax.dot_general(
            ins, w1t2_ref[...],
            dimension_numbers=(((1,), (0,)), ((), ())),
            preferred_element_type=jnp.float32)

    def post(t):
        tb = t[:, :HID * N].astype(jnp.bfloat16)
        hw = jnp.abs(tb) * wabs
        hw = hw[:, :1024] + hw[:, 1024:]
        hw = hw[:, :512] + hw[:, 512:]
        hw = hw[:, :256] + hw[:, 256:]
        hw = hw[:, :128] + hw[:, 128:]
        return (hw[:, :64] + hw[:, 64:]).astype(jnp.float32) + t[:, HID * N:]

    def finish(code, s, nd, outf, outb, ok):
        oh = lanef1 == code
        val = jnp.sum(jnp.where(oh[:, :n], s, 0.0),
                      axis=1, keepdims=True)
        wr = oh & nd & ok
        return (jnp.where(wr, val, outf),
                jnp.where(wr, val.astype(jnp.bfloat16), outb))

    zs = jnp.zeros((H, n), jnp.float32)
    zc = jnp.zeros((H, 1), jnp.bfloat16)

    def body(i, carry):
        oAf, oAb, oBf, oBb, sA, sB, cA, cB = carry
        ok = i > 0
        oAf, oAb = finish(cA, sA, ndA, oAf, oAb, ok)
        gA = gx_ref[i, :H]
        tA = mm(gA * oAb)
        oBf, oBb = finish(cB, sB, ndB, oBf, oBb, ok)
        gB = gx_ref[i, H:]
        tB = mm(gB * oBb)
        sA2 = post(tA)
        sB2 = post(tB)
        return (oAf, oAb, oBf, oBb, sA2, sB2,
                gA[:, NL - 1:], gB[:, NL - 1:])

    outb0 = outputs0.astype(jnp.bfloat16)
    oAf, oAb, oBf, oBb, sA, sB, cA, cB = jax.lax.fori_loop(
        0, n, body,
        (outputs0[:H], outb0[:H], outputs0[H:], outb0[H:], zs, zs, zc, zc))
    oAf, _ = finish(cA, sA, ndA, oAf, oAb, True)
    oBf, _ = finish(cB, sB, ndB, oBf, oBb, True)
    out_ref[:H] = oAf[:, :n]
    out_ref[H:] = oBf[:, :n]


def kernel(Z, A, order, do, W1, b1, W2, b2):
    m, n = Z.shape
    hid = W1.shape[1]
    u = 2.0 + jax.random.normal(jax.random.key(42), (m,), dtype=Z.dtype)

    kid = jnp.broadcast_to(jnp.arange(1, n + 1, dtype=Z.dtype)[None, :, None],
                           (m, n, 1))
    at_ext = jnp.concatenate(
        [jnp.swapaxes(A, 1, 2), Z[:, :, None], jnp.ones((m, n, 1), Z.dtype),
         kid], axis=2).astype(jnp.bfloat16)
    w1f = jnp.transpose(W1, (2, 1, 0)) * W2.T[None]
    bias = (b1.T * W2.T)[None]
    w1t2 = jnp.concatenate(
        [jnp.concatenate([w1f, bias], 0).reshape(n + 2, hid * n),
         jnp.zeros((1, hid * n), Z.dtype)], 0)
    extra = 0.505 * w1t2.reshape(NL, hid, n).sum(axis=1)
    extra = extra.at[n + 1, :].add(b2)
    w1t2 = jnp.concatenate([w1t2, extra], axis=1)
    w1t2 = w1t2.astype(jnp.bfloat16)
    wabs = (0.495 * jnp.sign(W2.T).reshape(1, hid * n)).astype(jnp.float32)
    b2r = b2.reshape(1, n)
    do2 = do.reshape(m, 1)
    u2 = u.reshape(m, 1)

    grid = m // M_BLK
    out = pl.pallas_call(
        _ncm_block_kernel,
        grid=(grid,),
        in_specs=[
            pl.BlockSpec((M_BLK, n, NL), lambda b: (b, 0, 0)),
            pl.BlockSpec((M_BLK, n), lambda b: (b, 0)),
            pl.BlockSpec((M_BLK, 1), lambda b: (b, 0)),
            pl.BlockSpec((M_BLK, 1), lambda b: (b, 0)),
            pl.BlockSpec((NL, hid * n + n), lambda b: (0, 0)),
            pl.BlockSpec((1, hid * n), lambda b: (0, 0)),
            pl.BlockSpec((1, n), lambda b: (0, 0)),
        ],
        out_specs=pl.BlockSpec((M_BLK, n), lambda b: (b, 0)),
        out_shape=jax.ShapeDtypeStruct((m, n), Z.dtype),
        scratch_shapes=[
            pltpu.VMEM((n, M_BLK, NL), jnp.bfloat16),
        ],
    )(at_ext, order, do2, u2, w1t2, wabs, b2r)
    return out


# speedup vs baseline: 1.0499x; 1.0499x over previous
import jax
import jax.numpy as jnp
from jax.experimental import pallas as pl
from jax.experimental.pallas import tpu as pltpu

M_BLK = 512
H = M_BLK // 2
N = 64
HID = 32
NL = N + 3


def _ncm_block_kernel(atext_ref, order_ref, do_ref, u_ref,
                      w1t2_ref, wabs_ref, b2_ref,
                      out_ref, gx_ref):
    m_blk = atext_ref.shape[0]
    n = N
    order = order_ref[...]
    do_b = do_ref[...]
    u_b = u_ref[...]

    iota_k = jax.lax.broadcasted_iota(jnp.int32, (m_blk, n, n), 2)
    onehot = (order[:, :, None] == iota_k).astype(jnp.bfloat16)
    gx = jax.lax.dot_general(
        onehot, atext_ref[...],
        dimension_numbers=(((2,), (1,)), ((0,), (0,))),
        preferred_element_type=jnp.float32)
    gx_ref[...] = jax.lax.transpose(gx.astype(jnp.bfloat16), (1, 0, 2))

    lane = jax.lax.broadcasted_iota(jnp.int32, (m_blk, NL), 1)
    wabs = wabs_ref[...].astype(jnp.bfloat16)
    lanef1 = (jax.lax.broadcasted_iota(jnp.int32, (1, NL), 1)
              + 1).astype(jnp.bfloat16)

    outputs0 = jnp.where(lane == do_b, u_b,
                         jnp.where(lane >= n, 1.0, 0.0))
    not_do = lane != do_b

    def body(i, carry):
        outf, outb = carry
        gef = gx_ref[i]
        ins = gef * outb
        t = jax.lax.dot_general(
            ins, w1t2_ref[...],
            dimension_numbers=(((1,), (0,)), ((), ())),
            preferred_element_type=jnp.float32)
        tb = t[:, :HID * N].astype(jnp.bfloat16)
        hw = jnp.abs(tb) * wabs
        hw = hw[:, :1024] + hw[:, 1024:]
        hw = hw[:, :512] + hw[:, 512:]
        hw = hw[:, :256] + hw[:, 256:]
        hw = hw[:, :128] + hw[:, 128:]
        s = (hw[:, :64] + hw[:, 64:]).astype(jnp.float32) + t[:, HID * N:]
        oh = lanef1 == gef[:, NL - 1:]
        val = jnp.sum(jnp.where(oh[:, :n], s, 0.0),
                      axis=1, keepdims=True)
        wr = oh & not_do
        return (jnp.where(wr, val, outf),
                jnp.where(wr, val.astype(jnp.bfloat16), outb))

    outf, _ = jax.lax.fori_loop(
        0, n, body, (outputs0, outputs0.astype(jnp.bfloat16)))
    out_ref[...] = outf[:, :n]


def kernel(Z, A, order, do, W1, b1, W2, b2):
    m, n = Z.shape
    hid = W1.shape[1]
    u = 2.0 + jax.random.normal(jax.random.key(42), (m,), dtype=Z.dtype)

    kid = jnp.broadcast_to(jnp.arange(1, n + 1, dtype=Z.dtype)[None, :, None],
                           (m, n, 1))
    at_ext = jnp.concatenate(
        [jnp.swapaxes(A, 1, 2), Z[:, :, None], jnp.ones((m, n, 1), Z.dtype),
         kid], axis=2).astype(jnp.bfloat16)
    w1f = jnp.transpose(W1, (2, 1, 0)) * W2.T[None]
    bias = (b1.T * W2.T)[None]
    w1t2 = jnp.concatenate(
        [jnp.concatenate([w1f, bias], 0).reshape(n + 2, hid * n),
         jnp.zeros((1, hid * n), Z.dtype)], 0)
    extra = 0.505 * w1t2.reshape(NL, hid, n).sum(axis=1)
    extra = extra.at[n + 1, :].add(b2)
    w1t2 = jnp.concatenate([w1t2, extra], axis=1)
    w1t2 = w1t2.astype(jnp.bfloat16)
    wabs = (0.495 * jnp.sign(W2.T).reshape(1, hid * n)).astype(jnp.float32)
    b2r = b2.reshape(1, n)
    do2 = do.reshape(m, 1)
    u2 = u.reshape(m, 1)

    grid = m // M_BLK
    out = pl.pallas_call(
        _ncm_block_kernel,
        grid=(grid,),
        in_specs=[
            pl.BlockSpec((M_BLK, n, NL), lambda b: (b, 0, 0)),
            pl.BlockSpec((M_BLK, n), lambda b: (b, 0)),
            pl.BlockSpec((M_BLK, 1), lambda b: (b, 0)),
            pl.BlockSpec((M_BLK, 1), lambda b: (b, 0)),
            pl.BlockSpec((NL, hid * n + n), lambda b: (0, 0)),
            pl.BlockSpec((1, hid * n), lambda b: (0, 0)),
            pl.BlockSpec((1, n), lambda b: (0, 0)),
        ],
        out_specs=pl.BlockSpec((M_BLK, n), lambda b: (b, 0)),
        out_shape=jax.ShapeDtypeStruct((m, n), Z.dtype),
        scratch_shapes=[
            pltpu.VMEM((n, M_BLK, NL), jnp.bfloat16),
        ],
    )(at_ext, order, do2, u2, w1t2, wabs, b2r)
    return out
